# baseline (speedup 1.0000x reference)
import jax
import jax.numpy as jnp
from jax import lax
from jax.experimental import pallas as pl
from jax.experimental.pallas import tpu as pltpu
from jax.experimental.pallas import tpu_sc as plsc

_N = 10000
_NP = 10240
_E = 320000
_NC = 2
_NS = 16
_GROUP = 128
_CHUNK = 20
_C0 = 4
_C1 = 4
_NCMAX = _C0
_TOTCHUNK = _NS * (_C0 + _C1)
_EP = _TOTCHUNK * _CHUNK * _GROUP
_DEPTH = 3
_RPT = _NP // _NS

_mesh = plsc.VectorSubcoreMesh(
    core_axis_name="c", subcore_axis_name="s", num_cores=_NC, num_subcores=_NS
)
_sc_params = pltpu.CompilerParams(use_tc_tiling_on_sc=False,
                                  needs_layout_passes=False)


def _worker_chunks():
    c = lax.axis_index("c")
    s = lax.axis_index("s")
    nch = jnp.where(c == 0, _C0, _C1)
    base = jnp.where(c == 0, s * _C0, _NS * _C0 + s * _C1)
    return c, s, nch, base


def _deg_body(sidx_hbm, didx_hbm, zeros_hbm, ones_hbm, out_hbm,
              sidx_v, didx_v, ones_v, degt_v, bcast_v, dego_sh, degi_sh,
              isem, ssem):
    c, s, nch, base = _worker_chunks()
    r0 = s * _RPT
    pltpu.sync_copy(ones_hbm, ones_v)
    pltpu.sync_copy(zeros_hbm.at[pl.ds(r0, _RPT)], dego_sh.at[pl.ds(r0, _RPT)])
    pltpu.sync_copy(zeros_hbm.at[pl.ds(r0, _RPT)], degi_sh.at[pl.ds(r0, _RPT)])
    plsc.subcore_barrier()

    idx_d = {}
    scat_d = {}
    for k in range(_NCMAX):
        d = k % _DEPTH
        idx_d[k] = (
            pltpu.make_async_copy(sidx_hbm.at[base + k], sidx_v.at[d], isem),
            pltpu.make_async_copy(didx_hbm.at[base + k], didx_v.at[d], isem),
        )
        cps = []
        for j in range(_CHUNK):
            cps.append(pltpu.make_async_copy(ones_v, dego_sh.at[sidx_v.at[d, j]], ssem))
            cps.append(pltpu.make_async_copy(ones_v, degi_sh.at[didx_v.at[d, j]], ssem))
        scat_d[k] = cps

    for cp in idx_d[0]:
        cp.start()
    for k in range(_NCMAX):
        @pl.when(k < nch)
        def _(k=k):
            for cp in idx_d[k]:
                cp.wait()
            if k >= 2:
                for cp in scat_d[k - 2]:
                    cp.wait()

        if k + 1 < _NCMAX:
            @pl.when(k + 1 < nch)
            def _(k=k):
                for cp in idx_d[k + 1]:
                    cp.start()

        @pl.when(k < nch)
        def _(k=k):
            for cp in scat_d[k]:
                cp.start(add=True)

    for k in range(_NCMAX):
        @pl.when((k < nch) & (k + 2 >= nch))
        def _(k=k):
            for cp in scat_d[k]:
                cp.wait()

    plsc.subcore_barrier()
    for acc_sh, oi in ((dego_sh, 0), (degi_sh, 1)):
        pltpu.sync_copy(acc_sh.at[pl.ds(r0, _RPT)], degt_v)

        def sbody(r, carry):
            bcast_v[r] = plsc.load_gather(degt_v, [jnp.full((16,), r, jnp.int32)])
            return carry

        lax.fori_loop(0, _RPT, sbody, 0)
        pltpu.sync_copy(bcast_v, out_hbm.at[c, oi, pl.ds(r0, _RPT)])


_deg_call = pl.kernel(
    _deg_body,
    out_type=jax.ShapeDtypeStruct((_NC, 2, _NP, 16), jnp.float32),
    mesh=_mesh,
    scratch_types=[
        pltpu.VMEM((_DEPTH, _CHUNK, _GROUP), jnp.int32),
        pltpu.VMEM((_DEPTH, _CHUNK, _GROUP), jnp.int32),
        pltpu.VMEM((_GROUP,), jnp.float32),
        pltpu.VMEM((_RPT,), jnp.float32),
        pltpu.VMEM((_RPT, 16), jnp.float32),
        pltpu.VMEM_SHARED((_NP,), jnp.float32),
        pltpu.VMEM_SHARED((_NP,), jnp.float32),
        pltpu.SemaphoreType.DMA,
        pltpu.SemaphoreType.DMA,
    ],
    compiler_params=_sc_params,
)


def _seg_body(table_hbm, sidx_hbm, didx_hbm, zeros_hbm, out_hbm,
              sidx_v, didx_v, rows_v, acc_sh, isem, gsem, ssem):
    c, s, nch, base = _worker_chunks()
    r0 = s * _RPT
    pltpu.sync_copy(zeros_hbm.at[pl.ds(r0, _RPT)], acc_sh.at[pl.ds(r0, _RPT)])
    plsc.subcore_barrier()

    idx_d = {}
    gat_d = {}
    scat_d = {}
    for k in range(_NCMAX):
        d = k % _DEPTH
        idx_d[k] = (
            pltpu.make_async_copy(sidx_hbm.at[base + k], sidx_v.at[d], isem),
            pltpu.make_async_copy(didx_hbm.at[base + k], didx_v.at[d], isem),
        )
        gat_d[k] = [pltpu.make_async_copy(table_hbm.at[sidx_v.at[d, j]],
                                          rows_v.at[k % 2, j], gsem)
                    for j in range(_CHUNK)]
        scat_d[k] = [pltpu.make_async_copy(rows_v.at[k % 2, j],
                                           acc_sh.at[didx_v.at[d, j]], ssem)
                     for j in range(_CHUNK)]

    for cp in idx_d[0]:
        cp.start()
    for k in range(_NCMAX):
        @pl.when(k < nch)
        def _(k=k):
            for cp in idx_d[k]:
                cp.wait()
            if k >= 2:
                for cp in scat_d[k - 2]:
                    cp.wait()
            for cp in gat_d[k]:
                cp.start()

        if k + 1 < _NCMAX:
            @pl.when(k + 1 < nch)
            def _(k=k):
                for cp in idx_d[k + 1]:
                    cp.start()

        @pl.when(k < nch)
        def _(k=k):
            for cp in gat_d[k]:
                cp.wait()
            for cp in scat_d[k]:
                cp.start(add=True)

    for k in range(_NCMAX):
        @pl.when((k < nch) & (k + 2 >= nch))
        def _(k=k):
            for cp in scat_d[k]:
                cp.wait()

    plsc.subcore_barrier()
    pltpu.sync_copy(acc_sh.at[pl.ds(r0, _RPT)], out_hbm.at[c, pl.ds(r0, _RPT)])


_seg_call = pl.kernel(
    _seg_body,
    out_type=jax.ShapeDtypeStruct((_NC, _NP, 16), jnp.float32),
    mesh=_mesh,
    scratch_types=[
        pltpu.VMEM((_DEPTH, _CHUNK, _GROUP), jnp.int32),
        pltpu.VMEM((_DEPTH, _CHUNK, _GROUP), jnp.int32),
        pltpu.VMEM((2, _CHUNK, _GROUP, 16), jnp.float32),
        pltpu.VMEM_SHARED((_NP, 16), jnp.float32),
        pltpu.SemaphoreType.DMA,
        pltpu.SemaphoreType.DMA,
        pltpu.SemaphoreType.DMA,
    ],
    compiler_params=_sc_params,
)


_NR = _NP // 8


def _p_body(f_ref, w1big_ref, p_ref):
    p_ref[0:_N // 8, :] = jnp.dot(f_ref[...], w1big_ref[...],
                                  preferred_element_type=jnp.float32)


_p_call = pl.pallas_call(
    _p_body,
    out_shape=jax.ShapeDtypeStruct((_NR, 128), jnp.float32),
)


def _norms_body(degp_ref, p_ref, w2_ref, wfc_ref, b2_ref, bfc_ref,
                h_ref, ns_ref, nd_ref, c_ref, d_ref):
    dego = degp_ref[0, 0] + degp_ref[1, 0]
    degi = degp_ref[0, 1] + degp_ref[1, 1]
    ns = jnp.where(dego > 0, lax.rsqrt(jnp.maximum(dego, 1.0)), 0.0)
    nd = jnp.where(degi > 0, lax.rsqrt(jnp.maximum(degi, 1.0)), 0.0)
    h_ref[...] = p_ref[...] * ns
    ns_ref[...] = ns
    nd_ref[...] = nd
    c_ref[...] = jnp.dot(w2_ref[...], wfc_ref[...],
                         preferred_element_type=jnp.float32)
    d_ref[...] = jnp.dot(b2_ref[...], wfc_ref[...],
                         preferred_element_type=jnp.float32) + bfc_ref[...]


_norms_call = pl.pallas_call(
    _norms_body,
    out_shape=(
        jax.ShapeDtypeStruct((_NR, 128), jnp.float32),
        jax.ShapeDtypeStruct((_NR, 128), jnp.float32),
        jax.ShapeDtypeStruct((_NR, 128), jnp.float32),
        jax.ShapeDtypeStruct((16, 3), jnp.float32),
        jax.ShapeDtypeStruct((1, 3), jnp.float32),
    ),
)


def _mid_body(aggp_ref, nd_ref, ns_ref, b1t_ref, x_ref):
    agg = aggp_ref[0] + aggp_ref[1]
    x_ref[...] = jnp.maximum(agg * nd_ref[...] + b1t_ref[...], 0.0) * ns_ref[...]


_mid_call = pl.pallas_call(
    _mid_body,
    out_shape=jax.ShapeDtypeStruct((_NR, 128), jnp.float32),
)


def _head_body(aggp_ref, nd_ref, cbig_ref, dbig_ref, o_ref):
    y = (aggp_ref[0] + aggp_ref[1]) * nd_ref[...]
    o_ref[...] = jnp.dot(y, cbig_ref[...],
                         preferred_element_type=jnp.float32) + dbig_ref[...]


_head_call = pl.pallas_call(
    _head_body,
    out_shape=jax.ShapeDtypeStruct((_NR, 24), jnp.float32),
)


def kernel(features, edge_index, W1, b1, W2, b2, Wfc, bfc):
    pad = _N + (jnp.arange(_EP - _E, dtype=jnp.int32) % (_NP - _N))
    ei = jnp.concatenate(
        [edge_index.astype(jnp.int32), jnp.stack([pad, pad])], axis=1)
    sidx = ei[0].reshape(_TOTCHUNK, _CHUNK, _GROUP)
    didx = ei[1].reshape(_TOTCHUNK, _CHUNK, _GROUP)
    z1 = jnp.zeros((_NP,), jnp.float32)
    z16 = jnp.zeros((_NP, 16), jnp.float32)
    ones = jnp.ones((_GROUP,), jnp.float32)

    r1 = jnp.arange(1024, dtype=jnp.int32)
    c1 = jnp.arange(128, dtype=jnp.int32)
    w1big = jnp.where((r1[:, None] // 128) == (c1[None, :] // 16),
                      jnp.tile(W1, (8, 8)), 0.0)

    f8 = features.reshape(_N // 8, 1024)
    p = _p_call(f8, w1big)
    degp = _deg_call(sidx, didx, z1, ones)
    h1s, ns, nd, cw, dv = _norms_call(
        degp.reshape(_NC, 2, _NR, 128), p,
        W2, Wfc, b2.reshape(1, 128), bfc.reshape(1, 3))

    agg1p = _seg_call(h1s.reshape(_NP, 16), sidx, didx, z16)
    x1s = _mid_call(agg1p.reshape(_NC, _NR, 128), nd, ns,
                    jnp.tile(b1, 8).reshape(1, 128))
    agg2p = _seg_call(x1s.reshape(_NP, 16), sidx, didx, z16)

    r2 = jnp.arange(128, dtype=jnp.int32)
    c2 = jnp.arange(24, dtype=jnp.int32)
    cbig = jnp.where((r2[:, None] // 16) == (c2[None, :] // 3),
                     jnp.tile(cw, (8, 8)), 0.0)
    out_big = _head_call(agg2p.reshape(_NC, _NR, 128), nd, cbig,
                         jnp.tile(dv, (1, 8)))
    return out_big[: _N // 8].reshape(_N, 3)


# speedup vs baseline: 1.0216x; 1.0216x over previous
import jax
import jax.numpy as jnp
from jax import lax
from jax.experimental import pallas as pl
from jax.experimental.pallas import tpu as pltpu
from jax.experimental.pallas import tpu_sc as plsc

_N = 10000
_NP = 10240
_E = 320000
_NC = 2
_NS = 16
_GROUP = 128
_CHUNK = 16
_C0 = 5
_C1 = 5
_NCMAX = _C0
_TOTCHUNK = _NS * (_C0 + _C1)
_EP = _TOTCHUNK * _CHUNK * _GROUP
_DEPTH = 3
_RPT = _NP // _NS

_mesh = plsc.VectorSubcoreMesh(
    core_axis_name="c", subcore_axis_name="s", num_cores=_NC, num_subcores=_NS
)
_sc_params = pltpu.CompilerParams(use_tc_tiling_on_sc=False,
                                  needs_layout_passes=False)


def _worker_chunks():
    c = lax.axis_index("c")
    s = lax.axis_index("s")
    nch = jnp.where(c == 0, _C0, _C1)
    base = jnp.where(c == 0, s * _C0, _NS * _C0 + s * _C1)
    return c, s, nch, base


def _deg_body(sidx_hbm, didx_hbm, zeros_hbm, ones_hbm, out_hbm,
              sidx_v, didx_v, ones_v, degt_v, bcast_v, dego_sh, degi_sh,
              isem, ssem):
    c, s, nch, base = _worker_chunks()
    r0 = s * _RPT
    pltpu.sync_copy(ones_hbm, ones_v)
    pltpu.sync_copy(zeros_hbm.at[pl.ds(r0, _RPT)], dego_sh.at[pl.ds(r0, _RPT)])
    pltpu.sync_copy(zeros_hbm.at[pl.ds(r0, _RPT)], degi_sh.at[pl.ds(r0, _RPT)])
    plsc.subcore_barrier()

    idx_d = {}
    scat_d = {}
    for k in range(_NCMAX):
        d = k % _DEPTH
        idx_d[k] = (
            pltpu.make_async_copy(sidx_hbm.at[base + k], sidx_v.at[d], isem),
            pltpu.make_async_copy(didx_hbm.at[base + k], didx_v.at[d], isem),
        )
        cps = []
        for j in range(_CHUNK):
            cps.append(pltpu.make_async_copy(ones_v, dego_sh.at[sidx_v.at[d, j]], ssem))
            cps.append(pltpu.make_async_copy(ones_v, degi_sh.at[didx_v.at[d, j]], ssem))
        scat_d[k] = cps

    for cp in idx_d[0]:
        cp.start()
    for k in range(_NCMAX):
        @pl.when(k < nch)
        def _(k=k):
            for cp in idx_d[k]:
                cp.wait()
            if k >= 2:
                for cp in scat_d[k - 2]:
                    cp.wait()

        if k + 1 < _NCMAX:
            @pl.when(k + 1 < nch)
            def _(k=k):
                for cp in idx_d[k + 1]:
                    cp.start()

        @pl.when(k < nch)
        def _(k=k):
            for cp in scat_d[k]:
                cp.start(add=True)

    for k in range(_NCMAX):
        @pl.when((k < nch) & (k + 2 >= nch))
        def _(k=k):
            for cp in scat_d[k]:
                cp.wait()

    plsc.subcore_barrier()
    for acc_sh, oi in ((dego_sh, 0), (degi_sh, 1)):
        pltpu.sync_copy(acc_sh.at[pl.ds(r0, _RPT)], degt_v)

        def sbody(r, carry):
            bcast_v[r] = plsc.load_gather(degt_v, [jnp.full((16,), r, jnp.int32)])
            return carry

        lax.fori_loop(0, _RPT, sbody, 0)
        pltpu.sync_copy(bcast_v, out_hbm.at[c, oi, pl.ds(r0, _RPT)])


_deg_call = pl.kernel(
    _deg_body,
    out_type=jax.ShapeDtypeStruct((_NC, 2, _NP, 16), jnp.float32),
    mesh=_mesh,
    scratch_types=[
        pltpu.VMEM((_DEPTH, _CHUNK, _GROUP), jnp.int32),
        pltpu.VMEM((_DEPTH, _CHUNK, _GROUP), jnp.int32),
        pltpu.VMEM((_GROUP,), jnp.float32),
        pltpu.VMEM((_RPT,), jnp.float32),
        pltpu.VMEM((_RPT, 16), jnp.float32),
        pltpu.VMEM_SHARED((_NP,), jnp.float32),
        pltpu.VMEM_SHARED((_NP,), jnp.float32),
        pltpu.SemaphoreType.DMA,
        pltpu.SemaphoreType.DMA,
    ],
    compiler_params=_sc_params,
)


def _seg_body(table_hbm, sidx_hbm, didx_hbm, zeros_hbm, out_hbm,
              sidx_v, didx_v, rows_v, acc_sh, isem, gsem, ssem):
    c, s, nch, base = _worker_chunks()
    r0 = s * _RPT
    pltpu.sync_copy(zeros_hbm.at[pl.ds(r0, _RPT)], acc_sh.at[pl.ds(r0, _RPT)])
    plsc.subcore_barrier()

    idx_d = {}
    gat_d = {}
    scat_d = {}
    for k in range(_NCMAX):
        d = k % _DEPTH
        idx_d[k] = (
            pltpu.make_async_copy(sidx_hbm.at[base + k], sidx_v.at[d], isem),
            pltpu.make_async_copy(didx_hbm.at[base + k], didx_v.at[d], isem),
        )
        gat_d[k] = [pltpu.make_async_copy(table_hbm.at[sidx_v.at[d, j]],
                                          rows_v.at[k % 2, j], gsem)
                    for j in range(_CHUNK)]
        scat_d[k] = [pltpu.make_async_copy(rows_v.at[k % 2, j],
                                           acc_sh.at[didx_v.at[d, j]], ssem)
                     for j in range(_CHUNK)]

    for cp in idx_d[0]:
        cp.start()
    for k in range(_NCMAX):
        @pl.when(k < nch)
        def _(k=k):
            for cp in idx_d[k]:
                cp.wait()
            if k >= 2:
                for cp in scat_d[k - 2]:
                    cp.wait()
            for cp in gat_d[k]:
                cp.start()

        if k + 1 < _NCMAX:
            @pl.when(k + 1 < nch)
            def _(k=k):
                for cp in idx_d[k + 1]:
                    cp.start()

        @pl.when(k < nch)
        def _(k=k):
            for cp in gat_d[k]:
                cp.wait()
            for cp in scat_d[k]:
                cp.start(add=True)

    for k in range(_NCMAX):
        @pl.when((k < nch) & (k + 2 >= nch))
        def _(k=k):
            for cp in scat_d[k]:
                cp.wait()

    plsc.subcore_barrier()
    pltpu.sync_copy(acc_sh.at[pl.ds(r0, _RPT)], out_hbm.at[c, pl.ds(r0, _RPT)])


_seg_call = pl.kernel(
    _seg_body,
    out_type=jax.ShapeDtypeStruct((_NC, _NP, 16), jnp.float32),
    mesh=_mesh,
    scratch_types=[
        pltpu.VMEM((_DEPTH, _CHUNK, _GROUP), jnp.int32),
        pltpu.VMEM((_DEPTH, _CHUNK, _GROUP), jnp.int32),
        pltpu.VMEM((2, _CHUNK, _GROUP, 16), jnp.float32),
        pltpu.VMEM_SHARED((_NP, 16), jnp.float32),
        pltpu.SemaphoreType.DMA,
        pltpu.SemaphoreType.DMA,
        pltpu.SemaphoreType.DMA,
    ],
    compiler_params=_sc_params,
)


_NR = _NP // 8


def _p_body(f_ref, w1big_ref, p_ref):
    p_ref[0:_N // 8, :] = jnp.dot(f_ref[...], w1big_ref[...],
                                  preferred_element_type=jnp.float32)


_p_call = pl.pallas_call(
    _p_body,
    out_shape=jax.ShapeDtypeStruct((_NR, 128), jnp.float32),
)


def _norms_body(degp_ref, p_ref, w2_ref, wfc_ref, b2_ref, bfc_ref,
                h_ref, ns_ref, nd_ref, c_ref, d_ref):
    dego = degp_ref[0, 0] + degp_ref[1, 0]
    degi = degp_ref[0, 1] + degp_ref[1, 1]
    ns = jnp.where(dego > 0, lax.rsqrt(jnp.maximum(dego, 1.0)), 0.0)
    nd = jnp.where(degi > 0, lax.rsqrt(jnp.maximum(degi, 1.0)), 0.0)
    h_ref[...] = p_ref[...] * ns
    ns_ref[...] = ns
    nd_ref[...] = nd
    c_ref[...] = jnp.dot(w2_ref[...], wfc_ref[...],
                         preferred_element_type=jnp.float32)
    d_ref[...] = jnp.dot(b2_ref[...], wfc_ref[...],
                         preferred_element_type=jnp.float32) + bfc_ref[...]


_norms_call = pl.pallas_call(
    _norms_body,
    out_shape=(
        jax.ShapeDtypeStruct((_NR, 128), jnp.float32),
        jax.ShapeDtypeStruct((_NR, 128), jnp.float32),
        jax.ShapeDtypeStruct((_NR, 128), jnp.float32),
        jax.ShapeDtypeStruct((16, 3), jnp.float32),
        jax.ShapeDtypeStruct((1, 3), jnp.float32),
    ),
)


def _mid_body(aggp_ref, nd_ref, ns_ref, b1t_ref, x_ref):
    agg = aggp_ref[0] + aggp_ref[1]
    x_ref[...] = jnp.maximum(agg * nd_ref[...] + b1t_ref[...], 0.0) * ns_ref[...]


_mid_call = pl.pallas_call(
    _mid_body,
    out_shape=jax.ShapeDtypeStruct((_NR, 128), jnp.float32),
)


def _head_body(aggp_ref, nd_ref, cbig_ref, dbig_ref, o_ref):
    y = (aggp_ref[0] + aggp_ref[1]) * nd_ref[...]
    o_ref[...] = jnp.dot(y, cbig_ref[...],
                         preferred_element_type=jnp.float32) + dbig_ref[...]


_head_call = pl.pallas_call(
    _head_body,
    out_shape=jax.ShapeDtypeStruct((_NR, 24), jnp.float32),
)


def kernel(features, edge_index, W1, b1, W2, b2, Wfc, bfc):
    pad = _N + (jnp.arange(_EP - _E, dtype=jnp.int32) % (_NP - _N))
    ei = jnp.concatenate(
        [edge_index.astype(jnp.int32), jnp.stack([pad, pad])], axis=1)
    sidx = ei[0].reshape(_TOTCHUNK, _CHUNK, _GROUP)
    didx = ei[1].reshape(_TOTCHUNK, _CHUNK, _GROUP)
    z1 = jnp.zeros((_NP,), jnp.float32)
    z16 = jnp.zeros((_NP, 16), jnp.float32)
    ones = jnp.ones((_GROUP,), jnp.float32)

    r1 = jnp.arange(1024, dtype=jnp.int32)
    c1 = jnp.arange(128, dtype=jnp.int32)
    w1big = jnp.where((r1[:, None] // 128) == (c1[None, :] // 16),
                      jnp.tile(W1, (8, 8)), 0.0)

    f8 = features.reshape(_N // 8, 1024)
    p = _p_call(f8, w1big)
    degp = _deg_call(sidx, didx, z1, ones)
    h1s, ns, nd, cw, dv = _norms_call(
        degp.reshape(_NC, 2, _NR, 128), p,
        W2, Wfc, b2.reshape(1, 128), bfc.reshape(1, 3))

    agg1p = _seg_call(h1s.reshape(_NP, 16), sidx, didx, z16)
    x1s = _mid_call(agg1p.reshape(_NC, _NR, 128), nd, ns,
                    jnp.tile(b1, 8).reshape(1, 128))
    agg2p = _seg_call(x1s.reshape(_NP, 16), sidx, didx, z16)

    r2 = jnp.arange(128, dtype=jnp.int32)
    c2 = jnp.arange(24, dtype=jnp.int32)
    cbig = jnp.where((r2[:, None] // 16) == (c2[None, :] // 3),
                     jnp.tile(cw, (8, 8)), 0.0)
    out_big = _head_call(agg2p.reshape(_NC, _NR, 128), nd, cbig,
                         jnp.tile(dv, (1, 8)))
    return out_big[: _N // 8].reshape(_N, 3)
